# baseline (speedup 1.0000x reference)
import functools

import jax
import jax.numpy as jnp
from jax import lax
from jax.experimental import pallas as pl
from jax.experimental.pallas import tpu as pltpu
from jax.experimental.pallas import tpu_sc as plsc

NB = 4096
NT = 200
DIM = 32
NC = 2
NS = 16
NW = NC * NS
G = 128
K = 5
N_OUTER = NT // K
L = 16

_mesh = plsc.VectorSubcoreMesh(core_axis_name="c", subcore_axis_name="s")


@functools.partial(
    pl.kernel,
    out_type=jax.ShapeDtypeStruct((NT, DIM, NB), jnp.float32),
    mesh=_mesh,
    compiler_params=pltpu.CompilerParams(use_tc_tiling_on_sc=False,
                                         needs_layout_passes=False),
    scratch_types=[
        pltpu.VMEM((NT, G), jnp.int32),
        pltpu.VMEM((K * G, DIM), jnp.float32),
        pltpu.VMEM((K * G, DIM), jnp.float32),
        pltpu.VMEM((K, DIM, G), jnp.float32),
        pltpu.VMEM((K, DIM, G), jnp.float32),
        pltpu.SemaphoreType.DMA,
        pltpu.SemaphoreType.DMA,
        pltpu.SemaphoreType.DMA,
        pltpu.SemaphoreType.DMA,
    ],
)
def _gather_kernel(idx_hbm, table_hbm, out_hbm, idx_all, g0, g1, t0, t1,
                   sg0, sg1, ss0, ss1):
    wid = lax.axis_index("s") * NC + lax.axis_index("c")
    b0 = wid * G

    pltpu.sync_copy(idx_hbm.at[:, pl.ds(b0, G)], idx_all)

    bufs = ((g0, t0, sg0, ss0), (g1, t1, sg1, ss1))
    iota = lax.iota(jnp.int32, L)
    bbs = [iota + (L * k) for k in range(G // L)]
    zeros = jnp.zeros((L,), jnp.int32)

    def fire_gathers(c, gbuf, sem):
        for j in range(K):
            pltpu.async_copy(
                table_hbm.at[idx_all.at[c * K + j]],
                gbuf.at[pl.ds(j * G, G)],
                sem,
            )

    def wait_gathers(gbuf, sem):
        pltpu.make_async_copy(table_hbm.at[pl.ds(0, K * G)], gbuf,
                              sem).wait()

    def transpose(gbuf, tbuf):
        for j in range(K):
            rows = [bbs[k] + (j * G) for k in range(G // L)]

            @pl.loop(0, DIM, unroll=8)
            def _d(d):
                dv = zeros + d
                for k in range(G // L):
                    val = plsc.load_gather(gbuf, [rows[k], dv])
                    tbuf[j, d, pl.ds(L * k, L)] = val

    def store(c, tbuf, sem):
        pltpu.async_copy(
            tbuf, out_hbm.at[pl.ds(c * K, K), :, pl.ds(b0, G)], sem)

    def wait_store(c, tbuf, sem):
        pltpu.make_async_copy(
            tbuf, out_hbm.at[pl.ds(c * K, K), :, pl.ds(b0, G)], sem).wait()

    fire_gathers(0, g0, sg0)
    fire_gathers(1, g1, sg1)

    @pl.loop(0, N_OUTER, step=2)
    def _pipe(i):
        for b in range(2):
            gbuf, tbuf, sg, ss = bufs[b]
            c = i + b
            wait_gathers(gbuf, sg)

            @pl.when(c >= 2)
            def _drain_store():
                wait_store(c - 2, tbuf, ss)

            transpose(gbuf, tbuf)

            @pl.when(c + 2 < N_OUTER)
            def _refire():
                fire_gathers(c + 2, gbuf, sg)

            store(c, tbuf, ss)

    for b in range(2):
        c = N_OUTER - 2 + b
        _, tbuf, _, ss = bufs[b]
        wait_store(c, tbuf, ss)


def kernel(token_ids, embedding_mat):
    out_t = _gather_kernel(token_ids.T, embedding_mat)
    return out_t.transpose(2, 0, 1)


# speedup vs baseline: 1.3292x; 1.3292x over previous
import functools

import jax
import jax.numpy as jnp
from jax import lax
from jax.experimental import pallas as pl
from jax.experimental.pallas import tpu as pltpu
from jax.experimental.pallas import tpu_sc as plsc

NB = 4096
NT = 200
DIM = 32
NC = 2
NS = 16
NW = NC * NS
G = 128
K = 10
N_OUTER = NT // K

_mesh = plsc.VectorSubcoreMesh(core_axis_name="c", subcore_axis_name="s")


@functools.partial(
    pl.kernel,
    out_type=jax.ShapeDtypeStruct((NT, NB, DIM), jnp.float32),
    mesh=_mesh,
    compiler_params=pltpu.CompilerParams(use_tc_tiling_on_sc=False),
    scratch_types=[
        pltpu.VMEM((NT, G), jnp.int32),
        pltpu.VMEM((K, G, DIM), jnp.float32),
        pltpu.VMEM((K, G, DIM), jnp.float32),
        pltpu.SemaphoreType.DMA,
        pltpu.SemaphoreType.DMA,
        pltpu.SemaphoreType.DMA,
        pltpu.SemaphoreType.DMA,
    ],
)
def _gather_kernel(idx_hbm, table_hbm, out_hbm, idx_all, rows0, rows1,
                   sg0, sg1, ss0, ss1):
    wid = lax.axis_index("s") * NC + lax.axis_index("c")
    b0 = wid * G

    pltpu.sync_copy(idx_hbm.at[:, pl.ds(b0, G)], idx_all)

    bufs = ((rows0, sg0, ss0), (rows1, sg1, ss1))

    def fire_gathers(c, buf, sem):
        for j in range(K):
            pltpu.async_copy(
                table_hbm.at[idx_all.at[c * K + j]],
                buf.at[j],
                sem,
            )

    def wait_gathers(buf, sem):
        pltpu.make_async_copy(out_hbm.at[pl.ds(0, K), pl.ds(0, G)], buf,
                              sem).wait()

    def store(c, buf, sem):
        return pltpu.async_copy(
            buf, out_hbm.at[pl.ds(c * K, K), pl.ds(b0, G)], sem)

    fire_gathers(0, rows0, sg0)
    fire_gathers(1, rows1, sg1)

    @pl.loop(0, N_OUTER - 2, step=2)
    def _pipe(i):
        for b in range(2):
            c = i + b
            buf, sg, ss = bufs[b]
            wait_gathers(buf, sg)
            store(c, buf, ss).wait()
            fire_gathers(c + 2, buf, sg)

    for b in range(2):
        c = N_OUTER - 2 + b
        buf, sg, ss = bufs[b]
        wait_gathers(buf, sg)
        store(c, buf, ss).wait()


def kernel(token_ids, embedding_mat):
    out_t = _gather_kernel(token_ids.T, embedding_mat)
    return out_t.transpose(1, 0, 2)
